# baseline (speedup 1.0000x reference)
import functools

import jax
import jax.numpy as jnp
from jax import lax
from jax.experimental import pallas as pl
from jax.experimental.pallas import tpu as pltpu
from jax.experimental.pallas import tpu_sc as plsc

N_NODES = 10000
N_EDGES = 320000
C_IN = 128
C_OUT = 128

D = 144
K = 64
NW = 32
CHUNKS = 160
E_PAD = NW * CHUNKS * K
N_PAD = 10016
RPT = N_PAD // 16
TRASH = N_NODES


def _proj_body(x_ref, w_ref, a_ref, bm_ref):
    x = x_ref[...]
    w = w_ref[...]
    w1 = w[:, :C_IN]
    w2 = w[:, C_IN:]
    dn = (((0,), (1,)), ((), ()))
    a_ref[...] = lax.dot_general(x, w1 - w2, dn, preferred_element_type=jnp.float32)
    bm = lax.dot_general(x, w2, dn, preferred_element_type=jnp.float32)
    bn = bm.shape[0]
    ones = jnp.ones((bn, 1), jnp.float32)
    zeros = jnp.zeros((bn, D - C_OUT - 1), jnp.float32)
    bm_ref[...] = jnp.concatenate([bm, ones, zeros], axis=1)


_proj = pl.pallas_call(
    _proj_body,
    out_shape=[
        jax.ShapeDtypeStruct((N_PAD, C_OUT), jnp.float32),
        jax.ShapeDtypeStruct((N_PAD, D), jnp.float32),
    ],
)


def _sc_body(table, g_hbm, r_hbm, z_hbm, out, g_v, r_v, rows_v, acc, sem):
    cid = lax.axis_index("c")
    sid = lax.axis_index("s")
    row0 = sid * RPT
    pltpu.sync_copy(z_hbm, acc.at[pl.ds(row0, RPT)])
    wid = sid * 2 + cid
    pltpu.sync_copy(g_hbm.at[wid], g_v)
    pltpu.sync_copy(r_hbm.at[wid], r_v)
    plsc.subcore_barrier()

    pltpu.async_copy(table.at[g_v.at[0]], rows_v.at[0], sem)

    def body(j, carry):
        p = lax.rem(j, 2)
        pltpu.make_async_copy(table.at[g_v.at[j]], rows_v.at[p], sem).wait()
        pltpu.async_copy(table.at[g_v.at[j + 1]], rows_v.at[1 - p], sem)
        pltpu.sync_copy(rows_v.at[p], acc.at[r_v.at[j]], add=True)
        return carry

    lax.fori_loop(0, CHUNKS - 1, body, 0)
    pl2 = (CHUNKS - 1) % 2
    pltpu.make_async_copy(table.at[g_v.at[CHUNKS - 1]], rows_v.at[pl2], sem).wait()
    pltpu.sync_copy(rows_v.at[pl2], acc.at[r_v.at[CHUNKS - 1]], add=True)
    plsc.subcore_barrier()
    pltpu.sync_copy(acc.at[pl.ds(row0, RPT)], out.at[cid, pl.ds(row0, RPT)])


@functools.cache
def _sc_scatter():
    return pl.kernel(
        _sc_body,
        mesh=plsc.VectorSubcoreMesh(core_axis_name="c", subcore_axis_name="s"),
        compiler_params=pltpu.CompilerParams(use_tc_tiling_on_sc=False),
        out_type=jax.ShapeDtypeStruct((2, N_PAD, D), jnp.float32),
        scratch_types=[
            pltpu.VMEM((CHUNKS, K), jnp.int32),
            pltpu.VMEM((CHUNKS, K), jnp.int32),
            pltpu.VMEM((2, K, D), jnp.float32),
            pltpu.VMEM_SHARED((N_PAD, D), jnp.float32),
            pltpu.SemaphoreType.DMA,
        ],
    )


def _comb_body(a_ref, s_ref, b_ref, o_ref):
    s = s_ref[0] + s_ref[1]
    sums = s[:, :C_OUT]
    cnt = s[:, C_OUT:C_OUT + 1]
    val = a_ref[...] + b_ref[...] + sums / jnp.maximum(cnt, 1.0)
    val = jnp.where(cnt > 0, val, 0.0)
    o_ref[...] = jnp.where(val > 0, val, 0.3 * val)


_comb = pl.pallas_call(
    _comb_body,
    out_shape=jax.ShapeDtypeStruct((N_PAD, C_OUT), jnp.float32),
)


def kernel(in_features, reduce_index, gather_index, W, b):
    x = in_features[0]
    x_pad = jnp.pad(x, ((0, 0), (0, N_PAD - N_NODES)))
    pad = jnp.full((E_PAD - N_EDGES,), TRASH, jnp.int32)
    g_idx = jnp.concatenate([gather_index, pad]).reshape(NW, CHUNKS, K)
    r_idx = jnp.concatenate([reduce_index, pad]).reshape(NW, CHUNKS, K)
    zeros = jnp.zeros((RPT, D), jnp.float32)

    a_t, table = _proj(x_pad, W)
    partials = _sc_scatter()(table, g_idx, r_idx, zeros)
    out_t = _comb(a_t, partials, b.reshape(1, C_OUT))
    return jnp.transpose(out_t[:N_NODES])[None]


# speedup vs baseline: 1.3126x; 1.3126x over previous
import functools

import jax
import jax.numpy as jnp
from jax import lax
from jax.experimental import pallas as pl
from jax.experimental.pallas import tpu as pltpu
from jax.experimental.pallas import tpu_sc as plsc

N_NODES = 10000
N_EDGES = 320000
C_IN = 128
C_OUT = 128

DT = 80
K = 128
CPT = 160
E_PAD = 16 * CPT * K
RING = 5
LOOK = 3
DRAIN = 2
RBLK = 32
N_PAD = 10016
RPT = N_PAD // 16
TRASH = N_NODES


def _proj_body(x_ref, w_ref, a_ref, t_ref):
    x = x_ref[...]
    w = w_ref[...]
    w1 = w[:, :C_IN]
    w2 = w[:, C_IN:]
    dn = (((0,), (1,)), ((), ()))
    a_ref[...] = lax.dot_general(x, w1 - w2, dn, preferred_element_type=jnp.float32)
    bm = lax.dot_general(x, w2, dn, preferred_element_type=jnp.float32)
    ones = jnp.ones((N_PAD, 1), jnp.float32)
    zeros = jnp.zeros((N_PAD, DT - 65), jnp.float32)
    t0 = jnp.concatenate([bm[:, :64], ones, zeros], axis=1)
    t1 = jnp.concatenate([bm[:, 64:], ones, zeros], axis=1)
    t_ref[...] = jnp.stack([t0, t1], axis=0)


_proj = pl.pallas_call(
    _proj_body,
    out_shape=[
        jax.ShapeDtypeStruct((N_PAD, C_OUT), jnp.float32),
        jax.ShapeDtypeStruct((2, N_PAD, DT), jnp.float32),
    ],
)


def _sc_body(table, g_hbm, r_hbm, z_hbm, out, g_v, r_v, rows_v, acc, sem_g, sem_s):
    cid = lax.axis_index("c")
    sid = lax.axis_index("s")
    row0 = sid * RPT
    pltpu.sync_copy(z_hbm, acc.at[pl.ds(row0, RPT)])
    pltpu.sync_copy(g_hbm.at[sid], g_v)
    pltpu.sync_copy(r_hbm.at[sid, pl.ds(0, RBLK)], r_v.at[0])
    plsc.subcore_barrier()

    my_table = table.at[cid]

    def issue_g(j):
        pltpu.async_copy(my_table.at[g_v.at[j]], rows_v.at[j % RING], sem_g)

    def wait_g(j):
        pltpu.make_async_copy(
            my_table.at[g_v.at[j]], rows_v.at[j % RING], sem_g).wait()

    def r_row(j):
        return r_v.at[(j // RBLK) % 2, j % RBLK]

    def issue_s(j):
        pltpu.async_copy(rows_v.at[j % RING], acc.at[r_row(j)], sem_s, add=True)

    def wait_s(j):
        pltpu.make_async_copy(rows_v.at[j % RING], acc.at[r_row(j)], sem_s).wait()

    for j in range(LOOK):
        issue_g(j)
    for j in range(DRAIN):
        wait_g(j)
        issue_s(j)
        issue_g(j + LOOK)

    def body(j, carry):
        @pl.when(lax.rem(j, RBLK) == RBLK // 8 + 1)
        def _():
            blk = lax.div(j, RBLK) + 1
            pltpu.sync_copy(
                r_hbm.at[sid, pl.ds(blk * RBLK, RBLK)],
                r_v.at[lax.rem(blk, 2)])
        wait_s(j - DRAIN)
        wait_g(j)
        issue_s(j)
        issue_g(j + LOOK)
        return carry

    lax.fori_loop(DRAIN, CPT - LOOK, body, 0)
    for j in range(CPT - LOOK, CPT):
        wait_s(j - DRAIN)
        wait_g(j)
        issue_s(j)
    for j in range(CPT - DRAIN, CPT):
        wait_s(j)
    plsc.subcore_barrier()
    pltpu.sync_copy(acc.at[pl.ds(row0, RPT)], out.at[cid, pl.ds(row0, RPT)])


@functools.cache
def _sc_scatter():
    return pl.kernel(
        _sc_body,
        mesh=plsc.VectorSubcoreMesh(core_axis_name="c", subcore_axis_name="s"),
        compiler_params=pltpu.CompilerParams(use_tc_tiling_on_sc=False),
        out_type=jax.ShapeDtypeStruct((2, N_PAD, DT), jnp.float32),
        scratch_types=[
            pltpu.VMEM((CPT, K), jnp.int32),
            pltpu.VMEM((2, RBLK, K), jnp.int32),
            pltpu.VMEM((RING, K, DT), jnp.float32),
            pltpu.VMEM_SHARED((N_PAD, DT), jnp.float32),
            pltpu.SemaphoreType.DMA,
            pltpu.SemaphoreType.DMA,
        ],
    )


def _comb_body(a_ref, s_ref, b_ref, o_ref):
    s0 = s_ref[0]
    s1 = s_ref[1]
    sums = jnp.concatenate([s0[:, :64], s1[:, :64]], axis=1)
    cnt = s0[:, 64:65]
    val = a_ref[...] + b_ref[...] + sums / jnp.maximum(cnt, 1.0)
    val = jnp.where(cnt > 0, val, 0.0)
    o_ref[...] = jnp.where(val > 0, val, 0.3 * val)


_comb = pl.pallas_call(
    _comb_body,
    out_shape=jax.ShapeDtypeStruct((N_PAD, C_OUT), jnp.float32),
)


def kernel(in_features, reduce_index, gather_index, W, b):
    x = in_features[0]
    x_pad = jnp.pad(x, ((0, 0), (0, N_PAD - N_NODES)))
    pad = jnp.full((E_PAD - N_EDGES,), TRASH, jnp.int32)
    g_idx = jnp.concatenate([gather_index, pad]).reshape(16, CPT, K)
    r_main = jnp.concatenate([reduce_index, pad]).reshape(16, CPT, K)
    r_idx = jnp.concatenate(
        [r_main, jnp.full((16, RBLK, K), TRASH, jnp.int32)], axis=1
    )
    zeros = jnp.zeros((RPT, DT), jnp.float32)

    a_t, table = _proj(x_pad, W)
    partials = _sc_scatter()(table, g_idx, r_idx, zeros)
    out_t = _comb(a_t, partials, b.reshape(1, C_OUT))
    return jnp.transpose(out_t[:N_NODES])[None]
